# Initial kernel scaffold; baseline (speedup 1.0000x reference)
import jax
import jax.numpy as jnp
from jax.experimental import pallas as pl


def kernel(feat_atom, feat_fg, edge_bond, efeat_bond, edge_overlap, efeat_overlap, refine_src, refine_dst, pool_src, pool_dst, eps_b, gW1_b, gb1_b, gbn_g_b, gbn_b_b, gW2_b, gb2_b, eps_o, gW1_o, gb1_o, gbn_g_o, gbn_b_o, gW2_o, gb2_o, r_fcW, r_attnl, r_attnr, r_resW, r_bias, r_bn_g, r_bn_b, r_outW, r_outb, p_fcW, p_attnl, p_attnr, p_resW, p_bias, p_bn_g, p_bn_b, p_outW, p_outb):
    raise NotImplementedError("write your pallas kernel here")


# speedup vs baseline: 6.4019x; 6.4019x over previous
import functools

import jax
import jax.numpy as jnp
from jax import lax
from jax.experimental import pallas as pl
from jax.experimental.pallas import tpu as pltpu
from jax.experimental.pallas import tpu_sc as plsc

H = 4
D = 128
NA = 10000
NF = 2000
EB = 320000
EO = 32000
ER = 50000
EP = 50000

NA_PAD = 10240
NF_PAD = 2048
EGAT_PAD = 51200

_MESH = dict(core_axis_name="c", subcore_axis_name="s")


def _zero_vmem2d(ref, nrows, ncols):
    def body(i, _):
        for j in range(ncols // 16):
            ref[i, pl.ds(j * 16, 16)] = jnp.zeros((16,), jnp.float32)
        return 0
    lax.fori_loop(0, nrows, body, 0)


def _make_gine(E, ND_PAD, CH):
    EPW = E // 32
    NCHUNK = EPW // CH
    RPT = ND_PAD // 16
    NZ = RPT // 64

    @functools.partial(
        pl.kernel,
        out_type=jax.ShapeDtypeStruct((2, ND_PAD, D), jnp.float32),
        mesh=plsc.VectorSubcoreMesh(**_MESH),
        compiler_params=pltpu.CompilerParams(needs_layout_passes=False,
                                             use_tc_tiling_on_sc=False,
                                             has_side_effects=True),
        scratch_types=[
            pltpu.VMEM((CH,), jnp.int32),
            pltpu.VMEM((CH,), jnp.int32),
            pltpu.VMEM((CH, D), jnp.float32),
            pltpu.VMEM((CH, D), jnp.float32),
            pltpu.VMEM((64, D), jnp.float32),
            pltpu.VMEM_SHARED((ND_PAD, D), jnp.float32),
            pltpu.SemaphoreType.DMA,
        ],
    )
    def gine(feat_hbm, src_hbm, dst_hbm, ef_hbm, out_hbm,
             src_v, dst_v, rows_v, ef_v, zeros_v, acc_sh, sem):
        c = lax.axis_index("c")
        s = lax.axis_index("s")
        base_row = s * RPT

        _zero_vmem2d(zeros_v, 64, D)
        for b in range(NZ):
            pltpu.sync_copy(zeros_v, acc_sh.at[pl.ds(base_row + b * 64, 64)])
        plsc.subcore_barrier()

        ebase = (c * 16 + s) * EPW

        def chunk(ci, _):
            off = ebase + ci * CH
            pltpu.sync_copy(src_hbm.at[pl.ds(off, CH)], src_v)
            pltpu.sync_copy(dst_hbm.at[pl.ds(off, CH)], dst_v)
            gcopy = pltpu.async_copy(feat_hbm.at[src_v], rows_v, sem)
            pltpu.sync_copy(ef_hbm.at[pl.ds(off, CH)], ef_v)
            gcopy.wait()

            def ebody(e, _):
                for j in range(D // 16):
                    sl = pl.ds(j * 16, 16)
                    rows_v[e, sl] = jnp.maximum(rows_v[e, sl] + ef_v[e, sl],
                                                0.0)
                return 0
            lax.fori_loop(0, CH, ebody, 0)
            pltpu.sync_copy(rows_v, acc_sh.at[dst_v], add=True)
            return 0

        lax.fori_loop(0, NCHUNK, chunk, 0)
        plsc.subcore_barrier()
        pltpu.sync_copy(acc_sh.at[pl.ds(base_row, RPT)],
                        out_hbm.at[c, pl.ds(base_row, RPT)])

    return gine


def _make_gat(NS, ND_ER, ND_PAD, CH, NHALF):
    EPT = EGAT_PAD // 16
    NCHUNK = EPT // CH
    RPT = ND_PAD // 16
    NZ = RPT // 64
    DH = D // NHALF

    @functools.partial(
        pl.kernel,
        out_type=(jax.ShapeDtypeStruct((NHALF, H, ND_PAD, DH), jnp.float32),
                  jax.ShapeDtypeStruct((2, ND_PAD, 16), jnp.float32)),
        mesh=plsc.VectorSubcoreMesh(**_MESH),
        compiler_params=pltpu.CompilerParams(needs_layout_passes=False,
                                             use_tc_tiling_on_sc=False,
                                             has_side_effects=True),
        scratch_types=[
            pltpu.VMEM((NS * H,), jnp.float32),
            pltpu.VMEM((ND_ER * H,), jnp.float32),
            pltpu.VMEM((CH,), jnp.int32),
            pltpu.VMEM((CH,), jnp.int32),
            pltpu.VMEM((CH, DH), jnp.float32),
            pltpu.VMEM((CH, DH), jnp.float32),
            pltpu.VMEM((CH, 16), jnp.float32),
            pltpu.VMEM((CH,), jnp.float32),
            pltpu.VMEM((64, DH), jnp.float32),
            pltpu.VMEM((64, 16), jnp.float32),
            pltpu.VMEM_SHARED((ND_PAD, DH), jnp.float32),
            pltpu.VMEM_SHARED((ND_PAD, 16), jnp.float32),
            pltpu.SemaphoreType.DMA,
        ],
    )
    def gat(el_hbm, er_hbm, src_hbm, dst_hbm, feat_hbm,
            featout_hbm, denout_hbm,
            el_v, er_v, src_v, dst_v, rows_v, w_v, denrows_v, ee_v,
            zeros_v, zden_v, acc_sh, den_sh, sem):
        c = lax.axis_index("c")
        s = lax.axis_index("s")
        base_row = s * RPT

        pltpu.sync_copy(el_hbm, el_v)
        pltpu.sync_copy(er_hbm, er_v)
        _zero_vmem2d(zeros_v, 64, DH)
        _zero_vmem2d(zden_v, 64, 16)
        for b in range(NZ):
            pltpu.sync_copy(zden_v, den_sh.at[pl.ds(base_row + b * 64, 64)])

        for p in range(2):
            h = c * 2 + p
            for half in range(NHALF):
                for b in range(NZ):
                    pltpu.sync_copy(zeros_v,
                                    acc_sh.at[pl.ds(base_row + b * 64, 64)])
                _zero_vmem2d(denrows_v, CH, 16)
                plsc.subcore_barrier()
                first = half == 0

                def chunk(ci, _):
                    off = s * EPT + ci * CH
                    pltpu.sync_copy(src_hbm.at[pl.ds(off, CH)], src_v)
                    pltpu.sync_copy(dst_hbm.at[pl.ds(off, CH)], dst_v)
                    gcopy = pltpu.async_copy(
                        feat_hbm.at[half].at[src_v], rows_v, sem)
                    for g in range(CH // 16):
                        sv = src_v[pl.ds(g * 16, 16)]
                        dv = dst_v[pl.ds(g * 16, 16)]
                        el = plsc.load_gather(el_v, [sv * H + h])
                        er = plsc.load_gather(er_v, [dv * H + h])
                        x = el + er
                        x = jnp.maximum(x, 0.2 * x)
                        ee = jnp.exp(x)
                        ee_v[pl.ds(g * 16, 16)] = ee
                        if first:
                            ridx = lax.iota(jnp.int32, 16) + g * 16
                            hvec = jnp.zeros((16,), jnp.int32) + h
                            plsc.store_scatter(denrows_v, [ridx, hvec], ee)
                    gcopy.wait()

                    def ebody(e, _):
                        scale = plsc.load_gather(
                            ee_v, [jnp.zeros((16,), jnp.int32) + e])
                        for j in range(DH // 16):
                            sl = pl.ds(j * 16, 16)
                            w_v[e, sl] = rows_v[e, sl] * scale
                        return 0
                    lax.fori_loop(0, CH, ebody, 0)
                    pltpu.sync_copy(w_v, acc_sh.at[dst_v], add=True)
                    if first:
                        pltpu.sync_copy(denrows_v, den_sh.at[dst_v],
                                        add=True)
                    return 0

                lax.fori_loop(0, NCHUNK, chunk, 0)
                plsc.subcore_barrier()
                pltpu.sync_copy(
                    acc_sh.at[pl.ds(base_row, RPT)],
                    featout_hbm.at[half].at[h, pl.ds(base_row, RPT)])

        plsc.subcore_barrier()
        pltpu.sync_copy(den_sh.at[pl.ds(base_row, RPT)],
                        denout_hbm.at[c, pl.ds(base_row, RPT)])

    return gat


def _tc_logits(feat_fg, feat_atom, r_fcW, r_attnl, r_attnr,
               p_fcW, p_attnl, p_attnr):
    def body(ffg, fat, rfc, ral, rar, pfc, pal, par,
             elr, err, elp, erp):
        def headmat(fcW, attn):
            cols = []
            for h in range(H):
                Wh = fcW[:, h * D:(h + 1) * D]
                vh = (Wh * attn[h:h + 1, :]).sum(axis=1, keepdims=True)
                cols.append(vh)
            return jnp.concatenate(cols, axis=1)

        elr[...] = ffg[...] @ headmat(rfc[...], ral[...])
        err[...] = jnp.concatenate(
            [fat[...] @ headmat(rfc[...], rar[...]),
             jnp.zeros((8, H), jnp.float32)], axis=0)
        elp[...] = fat[...] @ headmat(pfc[...], pal[...])
        erp[...] = jnp.concatenate(
            [ffg[...] @ headmat(pfc[...], par[...]),
             jnp.zeros((8, H), jnp.float32)], axis=0)

    return pl.pallas_call(
        body,
        out_shape=[
            jax.ShapeDtypeStruct((NF, H), jnp.float32),
            jax.ShapeDtypeStruct((NA + 8, H), jnp.float32),
            jax.ShapeDtypeStruct((NA, H), jnp.float32),
            jax.ShapeDtypeStruct((NF + 8, H), jnp.float32),
        ],
    )(feat_fg, feat_atom, r_fcW, r_attnl, r_attnr, p_fcW, p_attnl, p_attnr)


def _gine_mlp_a(feat, p0, p1, W1, b1, eps11, ND, BLK):
    grid = (ND // BLK,)

    def body(f_b, p0_b, p1_b, W1_r, b1_r, eps_r, x_b, st_b):
        i = pl.program_id(0)
        rst = (1.0 + eps_r[0, 0]) * f_b[...] + p0_b[...] + p1_b[...]
        x = rst @ W1_r[...] + b1_r[...]
        x_b[...] = x

        @pl.when(i == 0)
        def _():
            st_b[...] = jnp.zeros_like(st_b)
        st_b[...] += jnp.concatenate(
            [x.sum(0, keepdims=True), (x * x).sum(0, keepdims=True)], axis=0)

    return pl.pallas_call(
        body,
        grid=grid,
        in_specs=[
            pl.BlockSpec((BLK, D), lambda i: (i, 0)),
            pl.BlockSpec((BLK, D), lambda i: (i, 0)),
            pl.BlockSpec((BLK, D), lambda i: (i, 0)),
            pl.BlockSpec((D, 2 * D), lambda i: (0, 0)),
            pl.BlockSpec((1, 2 * D), lambda i: (0, 0)),
            pl.BlockSpec((1, 1), lambda i: (0, 0)),
        ],
        out_specs=[
            pl.BlockSpec((BLK, 2 * D), lambda i: (i, 0)),
            pl.BlockSpec((2, 2 * D), lambda i: (0, 0)),
        ],
        out_shape=[
            jax.ShapeDtypeStruct((ND, 2 * D), jnp.float32),
            jax.ShapeDtypeStruct((2, 2 * D), jnp.float32),
        ],
    )(feat, p0, p1, W1, b1, eps11)


def _gine_mlp_b(x, st, g1, be1, W2, b2, ND, BLK):
    grid = (ND // BLK,)

    def body(x_b, st_r, g_r, be_r, W2_r, b2_r, o_b):
        mu = st_r[0:1, :] / ND
        var = st_r[1:2, :] / ND - mu * mu
        xn = (x_b[...] - mu) * lax.rsqrt(var + 1e-5) * g_r[...] + be_r[...]
        o_b[...] = jnp.maximum(xn, 0.0) @ W2_r[...] + b2_r[...]

    return pl.pallas_call(
        body,
        grid=grid,
        in_specs=[
            pl.BlockSpec((BLK, 2 * D), lambda i: (i, 0)),
            pl.BlockSpec((2, 2 * D), lambda i: (0, 0)),
            pl.BlockSpec((1, 2 * D), lambda i: (0, 0)),
            pl.BlockSpec((1, 2 * D), lambda i: (0, 0)),
            pl.BlockSpec((2 * D, D), lambda i: (0, 0)),
            pl.BlockSpec((1, D), lambda i: (0, 0)),
        ],
        out_specs=pl.BlockSpec((BLK, D), lambda i: (i, 0)),
        out_shape=jax.ShapeDtypeStruct((ND, D), jnp.float32),
    )(x, st, g1, be1, W2, b2)


def _gat_mix_a(aggf_halves, den0, den1, featd, fcW, resW, bias, ND, BLK):
    grid = (ND // BLK,)
    nhalf = len(aggf_halves)
    dh = D // nhalf

    def body(*refs):
        a_bs = refs[:nhalf]
        d0_b, d1_b, f_b, fc_r, rw_r, bias_r, flat_b, st_b = refs[nhalf:]
        i = pl.program_id(0)
        den = d0_b[...] + d1_b[...]
        res = f_b[...] @ rw_r[...]
        outs = []
        for h in range(H):
            dd = den[:, h:h + 1] + 1e-9
            agg = jnp.concatenate([a_bs[k][h] for k in range(nhalf)], axis=1)
            outs.append((agg / dd) @ fc_r[:, h * D:(h + 1) * D])
        flat = jnp.concatenate(outs, axis=1) + res + bias_r[...]
        flat_b[...] = flat

        @pl.when(i == 0)
        def _():
            st_b[...] = jnp.zeros_like(st_b)
        st_b[...] += jnp.concatenate(
            [flat.sum(0, keepdims=True), (flat * flat).sum(0, keepdims=True)],
            axis=0)

    return pl.pallas_call(
        body,
        grid=grid,
        in_specs=[pl.BlockSpec((H, BLK, dh), lambda i: (0, i, 0))] * nhalf + [
            pl.BlockSpec((BLK, 16), lambda i: (i, 0)),
            pl.BlockSpec((BLK, 16), lambda i: (i, 0)),
            pl.BlockSpec((BLK, D), lambda i: (i, 0)),
            pl.BlockSpec((D, H * D), lambda i: (0, 0)),
            pl.BlockSpec((D, H * D), lambda i: (0, 0)),
            pl.BlockSpec((1, H * D), lambda i: (0, 0)),
        ],
        out_specs=[
            pl.BlockSpec((BLK, H * D), lambda i: (i, 0)),
            pl.BlockSpec((2, H * D), lambda i: (0, 0)),
        ],
        out_shape=[
            jax.ShapeDtypeStruct((ND, H * D), jnp.float32),
            jax.ShapeDtypeStruct((2, H * D), jnp.float32),
        ],
    )(*aggf_halves, den0, den1, featd, fcW, resW, bias)


def _gat_mix_b(flat, st, bng, bnb, outW, outb, other, ND, BLK):
    grid = (ND // BLK,)

    def body(x_b, st_r, g_r, be_r, W_r, b_r, o_other, o_b):
        mu = st_r[0:1, :] / ND
        var = st_r[1:2, :] / ND - mu * mu
        xn = (x_b[...] - mu) * lax.rsqrt(var + 1e-5) * g_r[...] + be_r[...]
        o_b[...] = (jnp.maximum(xn, 0.0) @ W_r[...] + b_r[...]
                    + o_other[...])

    return pl.pallas_call(
        body,
        grid=grid,
        in_specs=[
            pl.BlockSpec((BLK, H * D), lambda i: (i, 0)),
            pl.BlockSpec((2, H * D), lambda i: (0, 0)),
            pl.BlockSpec((1, H * D), lambda i: (0, 0)),
            pl.BlockSpec((1, H * D), lambda i: (0, 0)),
            pl.BlockSpec((H * D, D), lambda i: (0, 0)),
            pl.BlockSpec((1, D), lambda i: (0, 0)),
            pl.BlockSpec((BLK, D), lambda i: (i, 0)),
        ],
        out_specs=pl.BlockSpec((BLK, D), lambda i: (i, 0)),
        out_shape=jax.ShapeDtypeStruct((ND, D), jnp.float32),
    )(flat, st, bng, bnb, outW, outb, other)


_gine_bond = _make_gine(EB, NA_PAD, 80)
_gine_ovl = _make_gine(EO, NF_PAD, 40)
_gat_refine = _make_gat(NF, NA + 8, NA_PAD, 80, 2)
_gat_pool = _make_gat(NA, NF + 8, NF_PAD, 80, 1)


def kernel(feat_atom, feat_fg, edge_bond, efeat_bond, edge_overlap,
           efeat_overlap, refine_src, refine_dst, pool_src, pool_dst,
           eps_b, gW1_b, gb1_b, gbn_g_b, gbn_b_b, gW2_b, gb2_b,
           eps_o, gW1_o, gb1_o, gbn_g_o, gbn_b_o, gW2_o, gb2_o,
           r_fcW, r_attnl, r_attnr, r_resW, r_bias, r_bn_g, r_bn_b,
           r_outW, r_outb,
           p_fcW, p_attnl, p_attnr, p_resW, p_bias, p_bn_g, p_bn_b,
           p_outW, p_outb):
    el_r, er_r, el_p, er_p = _tc_logits(
        feat_fg, feat_atom, r_fcW, r_attnl, r_attnr, p_fcW, p_attnl, p_attnr)

    bond_part = _gine_bond(feat_atom, edge_bond[0], edge_bond[1], efeat_bond)

    tok = bond_part[0, 0, 0]
    feat_fg_d, tok = lax.optimization_barrier((feat_fg, tok))
    ovl_part = _gine_ovl(feat_fg_d, edge_overlap[0], edge_overlap[1],
                         efeat_overlap)

    npad = EGAT_PAD - ER
    rs = jnp.concatenate([refine_src, jnp.zeros((npad,), jnp.int32)])
    rd = jnp.concatenate([refine_dst, jnp.full((npad,), NA, jnp.int32)])
    ps = jnp.concatenate([pool_src, jnp.zeros((npad,), jnp.int32)])
    pd = jnp.concatenate([pool_dst, jnp.full((npad,), NF, jnp.int32)])

    ffg_halves = jnp.stack([feat_fg[:, :64], feat_fg[:, 64:]])
    tok = ovl_part[0, 0, 0]
    el_r_d, tok = lax.optimization_barrier((el_r, tok))
    ref_feat, ref_den = _gat_refine(
        el_r_d.reshape(-1), er_r.reshape(-1), rs, rd, ffg_halves)
    tok = ref_den[0, 0, 0]
    el_p_d, tok = lax.optimization_barrier((el_p, tok))
    pool_feat, pool_den = _gat_pool(
        el_p_d.reshape(-1), er_p.reshape(-1), ps, pd,
        feat_atom.reshape(1, NA, D))

    xb, stb = _gine_mlp_a(feat_atom, bond_part[0, :NA], bond_part[1, :NA],
                          gW1_b, gb1_b.reshape(1, -1), eps_b.reshape(1, 1),
                          NA, 1000)
    h_bond = _gine_mlp_b(xb, stb, gbn_g_b.reshape(1, -1),
                         gbn_b_b.reshape(1, -1), gW2_b, gb2_b.reshape(1, -1),
                         NA, 1000)
    xo, sto = _gine_mlp_a(feat_fg, ovl_part[0, :NF], ovl_part[1, :NF],
                          gW1_o, gb1_o.reshape(1, -1), eps_o.reshape(1, 1),
                          NF, 1000)
    h_ovl = _gine_mlp_b(xo, sto, gbn_g_o.reshape(1, -1),
                        gbn_b_o.reshape(1, -1), gW2_o, gb2_o.reshape(1, -1),
                        NF, 1000)

    fr, str_ = _gat_mix_a([ref_feat[0][:, :NA], ref_feat[1][:, :NA]],
                          ref_den[0, :NA], ref_den[1, :NA],
                          feat_atom, r_fcW, r_resW, r_bias.reshape(1, -1),
                          NA, 1000)
    out1 = _gat_mix_b(fr, str_, r_bn_g.reshape(1, -1), r_bn_b.reshape(1, -1),
                      r_outW, r_outb.reshape(1, -1), h_bond, NA, 1000)
    fp, stp = _gat_mix_a([pool_feat[0][:, :NF]], pool_den[0, :NF],
                         pool_den[1, :NF], feat_fg, p_fcW, p_resW,
                         p_bias.reshape(1, -1), NF, 1000)
    out2 = _gat_mix_b(fp, stp, p_bn_g.reshape(1, -1), p_bn_b.reshape(1, -1),
                      p_outW, p_outb.reshape(1, -1), h_ovl, NF, 1000)

    return out1, out2


# speedup vs baseline: 10.2627x; 1.6031x over previous
import functools

import jax
import jax.numpy as jnp
from jax import lax
from jax.experimental import pallas as pl
from jax.experimental.pallas import tpu as pltpu
from jax.experimental.pallas import tpu_sc as plsc

H = 4
D = 128
NA = 10000
NF = 2000
EB = 320000
EO = 32000
ER = 50000
EP = 50000

NA_PAD = 10240
NA_PAD_G = 10112
NF_PAD = 2048
EGAT_PAD = 51200

_MESH = dict(core_axis_name="c", subcore_axis_name="s")


def _zero_vmem2d(ref, nrows, ncols):
    def body(i, _):
        for j in range(ncols // 16):
            ref[i, pl.ds(j * 16, 16)] = jnp.zeros((16,), jnp.float32)
        return 0
    lax.fori_loop(0, nrows, body, 0)


def _make_gine(E, ND_PAD, CH):
    EPW = E // 32
    NCHUNK = EPW // CH
    RPT = ND_PAD // 16
    NZ = RPT // 64
    NZT = RPT - NZ * 64

    @functools.partial(
        pl.kernel,
        out_type=jax.ShapeDtypeStruct((2, ND_PAD, D), jnp.float32),
        mesh=plsc.VectorSubcoreMesh(**_MESH),
        compiler_params=pltpu.CompilerParams(needs_layout_passes=False,
                                             use_tc_tiling_on_sc=False,
                                             has_side_effects=True),
        scratch_types=[
            pltpu.VMEM((2, CH), jnp.int32),
            pltpu.VMEM((2, CH), jnp.int32),
            pltpu.VMEM((2, CH, D), jnp.float32),
            pltpu.VMEM((2, CH, D), jnp.float32),
            pltpu.VMEM((64, D), jnp.float32),
            pltpu.VMEM_SHARED((ND_PAD, D), jnp.float32),
            pltpu.SemaphoreType.DMA,
            pltpu.SemaphoreType.DMA,
            pltpu.SemaphoreType.DMA,
            pltpu.SemaphoreType.DMA,
        ],
    )
    def gine(feat_hbm, src_hbm, dst_hbm, ef_hbm, out_hbm,
             src_v, dst_v, rows_v, ef_v, zeros_v, acc_sh,
             semg0, semg1, seme0, seme1):
        c = lax.axis_index("c")
        s = lax.axis_index("s")
        base_row = s * RPT
        semg = (semg0, semg1)
        seme = (seme0, seme1)

        _zero_vmem2d(zeros_v, 64, D)
        for b in range(NZ):
            pltpu.sync_copy(zeros_v, acc_sh.at[pl.ds(base_row + b * 64, 64)])
        if NZT:
            pltpu.sync_copy(
                zeros_v.at[pl.ds(0, NZT)],
                acc_sh.at[pl.ds(base_row + NZ * 64, NZT)])
        plsc.subcore_barrier()

        ebase = (c * 16 + s) * EPW

        def issue(ci, b):
            off = ebase + ci * CH
            pltpu.sync_copy(src_hbm.at[pl.ds(off, CH)], src_v.at[b])
            pltpu.sync_copy(dst_hbm.at[pl.ds(off, CH)], dst_v.at[b])
            pltpu.async_copy(feat_hbm.at[src_v.at[b]], rows_v.at[b], semg[b])
            pltpu.async_copy(ef_hbm.at[pl.ds(off, CH)], ef_v.at[b], seme[b])

        def process(b):
            pltpu.make_async_copy(
                feat_hbm.at[src_v.at[b]], rows_v.at[b], semg[b]).wait()
            pltpu.make_async_copy(
                ef_hbm.at[pl.ds(0, CH)], ef_v.at[b], seme[b]).wait()

            def ebody(e, _):
                for j in range(D // 16):
                    sl = pl.ds(j * 16, 16)
                    rows_v[b, e, sl] = jnp.maximum(
                        rows_v[b, e, sl] + ef_v[b, e, sl], 0.0)
                return 0
            lax.fori_loop(0, CH, ebody, 0)
            pltpu.sync_copy(rows_v.at[b], acc_sh.at[dst_v.at[b]], add=True)

        issue(0, 0)

        def pair(k, _):
            ci0 = 2 * k
            issue(ci0 + 1, 1)
            process(0)

            @pl.when(ci0 + 2 < NCHUNK)
            def _():
                issue(ci0 + 2, 0)
            process(1)
            return 0

        lax.fori_loop(0, NCHUNK // 2, pair, 0)
        if NCHUNK % 2 == 1:
            process(0)
        plsc.subcore_barrier()
        pltpu.sync_copy(acc_sh.at[pl.ds(base_row, RPT)],
                        out_hbm.at[c, pl.ds(base_row, RPT)])

    return gine


def _make_gat(NS, ND_ER, ND_PAD, CH, NHALF):
    EPT = EGAT_PAD // 16
    NCHUNK = EPT // CH
    RPT = ND_PAD // 16
    NZ = RPT // 64
    DH = D // NHALF

    @functools.partial(
        pl.kernel,
        out_type=(jax.ShapeDtypeStruct((NHALF, H, ND_PAD, DH), jnp.float32),
                  jax.ShapeDtypeStruct((2, ND_PAD, 16), jnp.float32)),
        mesh=plsc.VectorSubcoreMesh(**_MESH),
        compiler_params=pltpu.CompilerParams(needs_layout_passes=False,
                                             use_tc_tiling_on_sc=False,
                                             has_side_effects=True),
        scratch_types=[
            pltpu.VMEM((NS * H,), jnp.float32),
            pltpu.VMEM((ND_ER * H,), jnp.float32),
            pltpu.VMEM((2, CH), jnp.int32),
            pltpu.VMEM((2, CH), jnp.int32),
            pltpu.VMEM((2, CH, DH), jnp.float32),
            pltpu.VMEM((CH, 16), jnp.float32),
            pltpu.VMEM((CH,), jnp.float32),
            pltpu.VMEM((64, DH), jnp.float32),
            pltpu.VMEM((64, 16), jnp.float32),
            pltpu.VMEM_SHARED((ND_PAD, DH), jnp.float32),
            pltpu.VMEM_SHARED((ND_PAD, 16), jnp.float32),
            pltpu.SemaphoreType.DMA,
            pltpu.SemaphoreType.DMA,
        ],
    )
    def gat(el_hbm, er_hbm, src_hbm, dst_hbm, feat_hbm,
            featout_hbm, denout_hbm,
            el_v, er_v, src_v, dst_v, rows_v, denrows_v, ee_v,
            zeros_v, zden_v, acc_sh, den_sh, semg0, semg1):
        c = lax.axis_index("c")
        s = lax.axis_index("s")
        base_row = s * RPT
        semg = (semg0, semg1)

        pltpu.sync_copy(el_hbm, el_v)
        pltpu.sync_copy(er_hbm, er_v)
        _zero_vmem2d(zeros_v, 64, DH)
        _zero_vmem2d(zden_v, 64, 16)
        for b in range(NZ):
            pltpu.sync_copy(zden_v, den_sh.at[pl.ds(base_row + b * 64, 64)])

        for p in range(2):
            h = c * 2 + p
            for half in range(NHALF):
                for b in range(NZ):
                    pltpu.sync_copy(zeros_v,
                                    acc_sh.at[pl.ds(base_row + b * 64, 64)])
                _zero_vmem2d(denrows_v, CH, 16)
                plsc.subcore_barrier()
                first = half == 0

                def issue(ci, b):
                    off = s * EPT + ci * CH
                    pltpu.sync_copy(src_hbm.at[pl.ds(off, CH)], src_v.at[b])
                    pltpu.sync_copy(dst_hbm.at[pl.ds(off, CH)], dst_v.at[b])
                    pltpu.async_copy(feat_hbm.at[half].at[src_v.at[b]],
                                     rows_v.at[b], semg[b])

                def process(b):
                    for g in range(CH // 16):
                        sv = src_v[b, pl.ds(g * 16, 16)]
                        dv = dst_v[b, pl.ds(g * 16, 16)]
                        el = plsc.load_gather(el_v, [sv * H + h])
                        er = plsc.load_gather(er_v, [dv * H + h])
                        x = el + er
                        x = jnp.maximum(x, 0.2 * x)
                        ee = jnp.exp(x)
                        ee_v[pl.ds(g * 16, 16)] = ee
                        if first:
                            ridx = lax.iota(jnp.int32, 16) + g * 16
                            hvec = jnp.zeros((16,), jnp.int32) + h
                            plsc.store_scatter(denrows_v, [ridx, hvec], ee)
                    pltpu.make_async_copy(
                        feat_hbm.at[half].at[src_v.at[b]],
                        rows_v.at[b], semg[b]).wait()

                    def ebody(e, _):
                        scale = plsc.load_gather(
                            ee_v, [jnp.zeros((16,), jnp.int32) + e])
                        for j in range(DH // 16):
                            sl = pl.ds(j * 16, 16)
                            rows_v[b, e, sl] = rows_v[b, e, sl] * scale
                        return 0
                    lax.fori_loop(0, CH, ebody, 0)
                    pltpu.sync_copy(rows_v.at[b], acc_sh.at[dst_v.at[b]],
                                    add=True)
                    if first:
                        pltpu.sync_copy(denrows_v, den_sh.at[dst_v.at[b]],
                                        add=True)

                issue(0, 0)

                def pair(k, _):
                    ci0 = 2 * k
                    issue(ci0 + 1, 1)
                    process(0)

                    @pl.when(ci0 + 2 < NCHUNK)
                    def _():
                        issue(ci0 + 2, 0)
                    process(1)
                    return 0

                lax.fori_loop(0, NCHUNK // 2, pair, 0)
                if NCHUNK % 2 == 1:
                    process(0)
                plsc.subcore_barrier()
                pltpu.sync_copy(
                    acc_sh.at[pl.ds(base_row, RPT)],
                    featout_hbm.at[half].at[h, pl.ds(base_row, RPT)])

        plsc.subcore_barrier()
        pltpu.sync_copy(den_sh.at[pl.ds(base_row, RPT)],
                        denout_hbm.at[c, pl.ds(base_row, RPT)])

    return gat


def _tc_logits(feat_fg, feat_atom, r_fcW, r_attnl, r_attnr,
               p_fcW, p_attnl, p_attnr):
    def body(ffg, fat, rfc, ral, rar, pfc, pal, par,
             elr, err, elp, erp):
        def headmat(fcW, attn):
            cols = []
            for h in range(H):
                Wh = fcW[:, h * D:(h + 1) * D]
                vh = (Wh * attn[h:h + 1, :]).sum(axis=1, keepdims=True)
                cols.append(vh)
            return jnp.concatenate(cols, axis=1)

        elr[...] = ffg[...] @ headmat(rfc[...], ral[...])
        err[...] = jnp.concatenate(
            [fat[...] @ headmat(rfc[...], rar[...]),
             jnp.zeros((8, H), jnp.float32)], axis=0)
        elp[...] = fat[...] @ headmat(pfc[...], pal[...])
        erp[...] = jnp.concatenate(
            [ffg[...] @ headmat(pfc[...], par[...]),
             jnp.zeros((8, H), jnp.float32)], axis=0)

    return pl.pallas_call(
        body,
        out_shape=[
            jax.ShapeDtypeStruct((NF, H), jnp.float32),
            jax.ShapeDtypeStruct((NA + 8, H), jnp.float32),
            jax.ShapeDtypeStruct((NA, H), jnp.float32),
            jax.ShapeDtypeStruct((NF + 8, H), jnp.float32),
        ],
    )(feat_fg, feat_atom, r_fcW, r_attnl, r_attnr, p_fcW, p_attnl, p_attnr)


def _gine_mlp_a(feat, p0, p1, W1, b1, eps11, ND, BLK):
    grid = (ND // BLK,)

    def body(f_b, p0_b, p1_b, W1_r, b1_r, eps_r, x_b, st_b):
        i = pl.program_id(0)
        rst = (1.0 + eps_r[0, 0]) * f_b[...] + p0_b[...] + p1_b[...]
        x = rst @ W1_r[...] + b1_r[...]
        x_b[...] = x

        @pl.when(i == 0)
        def _():
            st_b[...] = jnp.zeros_like(st_b)
        st_b[...] += jnp.concatenate(
            [x.sum(0, keepdims=True), (x * x).sum(0, keepdims=True)], axis=0)

    return pl.pallas_call(
        body,
        grid=grid,
        in_specs=[
            pl.BlockSpec((BLK, D), lambda i: (i, 0)),
            pl.BlockSpec((BLK, D), lambda i: (i, 0)),
            pl.BlockSpec((BLK, D), lambda i: (i, 0)),
            pl.BlockSpec((D, 2 * D), lambda i: (0, 0)),
            pl.BlockSpec((1, 2 * D), lambda i: (0, 0)),
            pl.BlockSpec((1, 1), lambda i: (0, 0)),
        ],
        out_specs=[
            pl.BlockSpec((BLK, 2 * D), lambda i: (i, 0)),
            pl.BlockSpec((2, 2 * D), lambda i: (0, 0)),
        ],
        out_shape=[
            jax.ShapeDtypeStruct((ND, 2 * D), jnp.float32),
            jax.ShapeDtypeStruct((2, 2 * D), jnp.float32),
        ],
    )(feat, p0, p1, W1, b1, eps11)


def _gine_mlp_b(x, st, g1, be1, W2, b2, ND, BLK):
    grid = (ND // BLK,)

    def body(x_b, st_r, g_r, be_r, W2_r, b2_r, o_b):
        mu = st_r[0:1, :] / ND
        var = st_r[1:2, :] / ND - mu * mu
        xn = (x_b[...] - mu) * lax.rsqrt(var + 1e-5) * g_r[...] + be_r[...]
        o_b[...] = jnp.maximum(xn, 0.0) @ W2_r[...] + b2_r[...]

    return pl.pallas_call(
        body,
        grid=grid,
        in_specs=[
            pl.BlockSpec((BLK, 2 * D), lambda i: (i, 0)),
            pl.BlockSpec((2, 2 * D), lambda i: (0, 0)),
            pl.BlockSpec((1, 2 * D), lambda i: (0, 0)),
            pl.BlockSpec((1, 2 * D), lambda i: (0, 0)),
            pl.BlockSpec((2 * D, D), lambda i: (0, 0)),
            pl.BlockSpec((1, D), lambda i: (0, 0)),
        ],
        out_specs=pl.BlockSpec((BLK, D), lambda i: (i, 0)),
        out_shape=jax.ShapeDtypeStruct((ND, D), jnp.float32),
    )(x, st, g1, be1, W2, b2)


def _gat_mix_a(aggf_halves, den0, den1, featd, fcW, resW, bias, ND, BLK):
    grid = (ND // BLK,)
    nhalf = len(aggf_halves)
    dh = D // nhalf

    def body(*refs):
        a_bs = refs[:nhalf]
        d0_b, d1_b, f_b, fc_r, rw_r, bias_r, flat_b, st_b = refs[nhalf:]
        i = pl.program_id(0)
        den = d0_b[...] + d1_b[...]
        res = f_b[...] @ rw_r[...]
        outs = []
        for h in range(H):
            dd = den[:, h:h + 1] + 1e-9
            agg = jnp.concatenate([a_bs[k][h] for k in range(nhalf)], axis=1)
            outs.append((agg / dd) @ fc_r[:, h * D:(h + 1) * D])
        flat = jnp.concatenate(outs, axis=1) + res + bias_r[...]
        flat_b[...] = flat

        @pl.when(i == 0)
        def _():
            st_b[...] = jnp.zeros_like(st_b)
        st_b[...] += jnp.concatenate(
            [flat.sum(0, keepdims=True), (flat * flat).sum(0, keepdims=True)],
            axis=0)

    return pl.pallas_call(
        body,
        grid=grid,
        in_specs=[pl.BlockSpec((H, BLK, dh), lambda i: (0, i, 0))] * nhalf + [
            pl.BlockSpec((BLK, 16), lambda i: (i, 0)),
            pl.BlockSpec((BLK, 16), lambda i: (i, 0)),
            pl.BlockSpec((BLK, D), lambda i: (i, 0)),
            pl.BlockSpec((D, H * D), lambda i: (0, 0)),
            pl.BlockSpec((D, H * D), lambda i: (0, 0)),
            pl.BlockSpec((1, H * D), lambda i: (0, 0)),
        ],
        out_specs=[
            pl.BlockSpec((BLK, H * D), lambda i: (i, 0)),
            pl.BlockSpec((2, H * D), lambda i: (0, 0)),
        ],
        out_shape=[
            jax.ShapeDtypeStruct((ND, H * D), jnp.float32),
            jax.ShapeDtypeStruct((2, H * D), jnp.float32),
        ],
    )(*aggf_halves, den0, den1, featd, fcW, resW, bias)


def _gat_mix_b(flat, st, bng, bnb, outW, outb, other, ND, BLK):
    grid = (ND // BLK,)

    def body(x_b, st_r, g_r, be_r, W_r, b_r, o_other, o_b):
        mu = st_r[0:1, :] / ND
        var = st_r[1:2, :] / ND - mu * mu
        xn = (x_b[...] - mu) * lax.rsqrt(var + 1e-5) * g_r[...] + be_r[...]
        o_b[...] = (jnp.maximum(xn, 0.0) @ W_r[...] + b_r[...]
                    + o_other[...])

    return pl.pallas_call(
        body,
        grid=grid,
        in_specs=[
            pl.BlockSpec((BLK, H * D), lambda i: (i, 0)),
            pl.BlockSpec((2, H * D), lambda i: (0, 0)),
            pl.BlockSpec((1, H * D), lambda i: (0, 0)),
            pl.BlockSpec((1, H * D), lambda i: (0, 0)),
            pl.BlockSpec((H * D, D), lambda i: (0, 0)),
            pl.BlockSpec((1, D), lambda i: (0, 0)),
            pl.BlockSpec((BLK, D), lambda i: (i, 0)),
        ],
        out_specs=pl.BlockSpec((BLK, D), lambda i: (i, 0)),
        out_shape=jax.ShapeDtypeStruct((ND, D), jnp.float32),
    )(flat, st, bng, bnb, outW, outb, other)


_gine_bond = _make_gine(EB, NA_PAD_G, 80)
_gine_ovl = _make_gine(EO, NF_PAD, 40)
_gat_refine = _make_gat(NF, NA + 8, NA_PAD, 128, 2)
_gat_pool = _make_gat(NA, NF + 8, NF_PAD, 128, 1)


def kernel(feat_atom, feat_fg, edge_bond, efeat_bond, edge_overlap,
           efeat_overlap, refine_src, refine_dst, pool_src, pool_dst,
           eps_b, gW1_b, gb1_b, gbn_g_b, gbn_b_b, gW2_b, gb2_b,
           eps_o, gW1_o, gb1_o, gbn_g_o, gbn_b_o, gW2_o, gb2_o,
           r_fcW, r_attnl, r_attnr, r_resW, r_bias, r_bn_g, r_bn_b,
           r_outW, r_outb,
           p_fcW, p_attnl, p_attnr, p_resW, p_bias, p_bn_g, p_bn_b,
           p_outW, p_outb):
    el_r, er_r, el_p, er_p = _tc_logits(
        feat_fg, feat_atom, r_fcW, r_attnl, r_attnr, p_fcW, p_attnl, p_attnr)

    bond_part = _gine_bond(feat_atom, edge_bond[0], edge_bond[1], efeat_bond)

    tok = bond_part[0, 0, 0]
    feat_fg_d, tok = lax.optimization_barrier((feat_fg, tok))
    ovl_part = _gine_ovl(feat_fg_d, edge_overlap[0], edge_overlap[1],
                         efeat_overlap)

    npad = EGAT_PAD - ER
    rs = jnp.concatenate([refine_src, jnp.zeros((npad,), jnp.int32)])
    rd = jnp.concatenate([refine_dst, jnp.full((npad,), NA, jnp.int32)])
    ps = jnp.concatenate([pool_src, jnp.zeros((npad,), jnp.int32)])
    pd = jnp.concatenate([pool_dst, jnp.full((npad,), NF, jnp.int32)])

    ffg_halves = jnp.stack([feat_fg[:, :64], feat_fg[:, 64:]])
    tok = ovl_part[0, 0, 0]
    el_r_d, tok = lax.optimization_barrier((el_r, tok))
    ref_feat, ref_den = _gat_refine(
        el_r_d.reshape(-1), er_r.reshape(-1), rs, rd, ffg_halves)
    tok = ref_den[0, 0, 0]
    el_p_d, tok = lax.optimization_barrier((el_p, tok))
    pool_feat, pool_den = _gat_pool(
        el_p_d.reshape(-1), er_p.reshape(-1), ps, pd,
        feat_atom.reshape(1, NA, D))

    xb, stb = _gine_mlp_a(feat_atom, bond_part[0, :NA], bond_part[1, :NA],
                          gW1_b, gb1_b.reshape(1, -1), eps_b.reshape(1, 1),
                          NA, 1000)
    h_bond = _gine_mlp_b(xb, stb, gbn_g_b.reshape(1, -1),
                         gbn_b_b.reshape(1, -1), gW2_b, gb2_b.reshape(1, -1),
                         NA, 1000)
    xo, sto = _gine_mlp_a(feat_fg, ovl_part[0, :NF], ovl_part[1, :NF],
                          gW1_o, gb1_o.reshape(1, -1), eps_o.reshape(1, 1),
                          NF, 1000)
    h_ovl = _gine_mlp_b(xo, sto, gbn_g_o.reshape(1, -1),
                        gbn_b_o.reshape(1, -1), gW2_o, gb2_o.reshape(1, -1),
                        NF, 1000)

    fr, str_ = _gat_mix_a([ref_feat[0][:, :NA], ref_feat[1][:, :NA]],
                          ref_den[0, :NA], ref_den[1, :NA],
                          feat_atom, r_fcW, r_resW, r_bias.reshape(1, -1),
                          NA, 1000)
    out1 = _gat_mix_b(fr, str_, r_bn_g.reshape(1, -1), r_bn_b.reshape(1, -1),
                      r_outW, r_outb.reshape(1, -1), h_bond, NA, 1000)
    fp, stp = _gat_mix_a([pool_feat[0][:, :NF]], pool_den[0, :NF],
                         pool_den[1, :NF], feat_fg, p_fcW, p_resW,
                         p_bias.reshape(1, -1), NF, 1000)
    out2 = _gat_mix_b(fp, stp, p_bn_g.reshape(1, -1), p_bn_b.reshape(1, -1),
                      p_outW, p_outb.reshape(1, -1), h_ovl, NF, 1000)

    return out1, out2
